# baseline (speedup 1.0000x reference)
import functools

import jax
import jax.numpy as jnp
from jax import lax
from jax.experimental import pallas as pl
from jax.experimental.pallas import tpu as pltpu
from jax.experimental.pallas import tpu_sc as plsc

D = 128
NC = 2
NS = 16
NW = NC * NS
CHUNK = 128
LANES = 16

NBUF = 2
SUB = 4
SUBR = CHUNK // SUB
SUPER = 8
SEDGES = SUPER * CHUNK
HROWS = CHUNK // 2

_PCOL = [0] * D
for _g in range(D // 32):
    for _i in range(LANES):
        _PCOL[32 * _g + 2 * _i] = 32 * _g + _i
        _PCOL[32 * _g + 2 * _i + 1] = 32 * _g + LANES + _i


def _mm_body(x_ref, w_ref, o_ref):
    o_ref[...] = jnp.dot(
        x_ref[...], w_ref[...],
        preferred_element_type=jnp.float32).astype(jnp.bfloat16)


def _matmul_bf16(x, W):
    M = x.shape[0]
    BM = 1000
    return pl.pallas_call(
        _mm_body,
        grid=(M // BM,),
        in_specs=[pl.BlockSpec((BM, D), lambda i: (i, 0)),
                  pl.BlockSpec((D, D), lambda i: (0, 0))],
        out_specs=pl.BlockSpec((BM, D), lambda i: (i, 0)),
        out_shape=jax.ShapeDtypeStruct((M, D), jnp.bfloat16),
    )(x, W)


def _fin_body(p_ref, b_ref, o_ref):
    o_ref[...] = jnp.maximum(p_ref[0] + p_ref[1] + b_ref[...], 0.0)


def _finalize(partials, b, N):
    BM = 1000
    return pl.pallas_call(
        _fin_body,
        grid=(N // BM,),
        in_specs=[pl.BlockSpec((2, BM, D), lambda i: (0, i, 0)),
                  pl.BlockSpec((1, D), lambda i: (0, 0))],
        out_specs=pl.BlockSpec((BM, D), lambda i: (i, 0)),
        out_shape=jax.ShapeDtypeStruct((N, D), jnp.float32),
    )(partials, b.reshape(1, D))


def _spmm(support, src, dsth, vals, e_per_w):
    N = support.shape[0]
    n_chunks = e_per_w // CHUNK
    n_super = n_chunks // SUPER
    rows_per_tile = ((N + NS - 1) // NS + 7) // 8 * 8
    NP = rows_per_tile * NS

    mesh = plsc.VectorSubcoreMesh(core_axis_name="c", subcore_axis_name="s")

    @functools.partial(
        pl.kernel,
        mesh=mesh,
        compiler_params=pltpu.CompilerParams(
            needs_layout_passes=False, use_tc_tiling_on_sc=False),
        out_type=jax.ShapeDtypeStruct((NC, NP, D), jnp.float32),
        scratch_types=[
            pltpu.VMEM_SHARED((NP, D), jnp.float32),
            pltpu.VMEM((SEDGES,), jnp.int32),
            pltpu.VMEM((SUPER * 2, 1, HROWS), jnp.int32),
            pltpu.VMEM((SEDGES + LANES,), jnp.float32),
        ] + [pltpu.VMEM((CHUNK, D // 2), jnp.int32) for _ in range(NBUF)]
          + [pltpu.VMEM((HROWS, D), jnp.float32) for _ in range(2)]
          + [pltpu.SemaphoreType.DMA
             for _ in range(NBUF * SUB + 2 + 3)],
    )
    def k(support_hbm, src_hbm, dsth_hbm, vals_hbm, out_hbm,
          acc, src_v, dsth_v, vals_v, *bufs_sems):
        rowsb = bufs_sems[:NBUF]
        hstag = bufs_sems[NBUF:NBUF + 2]
        gsem = bufs_sems[NBUF + 2:NBUF + 2 + NBUF * SUB]
        qssem = bufs_sems[NBUF + 2 + NBUF * SUB:NBUF + 4 + NBUF * SUB]
        isem = bufs_sems[NBUF + 4 + NBUF * SUB:]
        c = lax.axis_index("c")
        s = lax.axis_index("s")
        wid = c * NS + s

        def zrow(r, carry):
            for g in range(D // LANES):
                hstag[0][r, pl.ds(g * LANES, LANES)] = jnp.zeros(
                    (LANES,), jnp.float32)
            return carry
        lax.fori_loop(0, HROWS, zrow, 0)
        row0 = s * rows_per_tile
        nfull = rows_per_tile // HROWS
        rem = rows_per_tile % HROWS
        for i in range(nfull):
            pltpu.sync_copy(hstag[0], acc.at[pl.ds(row0 + i * HROWS, HROWS)])
        if rem:
            pltpu.sync_copy(hstag[0].at[pl.ds(0, rem)],
                            acc.at[pl.ds(row0 + nfull * HROWS, rem)])
        plsc.subcore_barrier()

        cbase = wid * n_chunks

        def half_scatter_desc(x):
            return pltpu.make_async_copy(
                hstag[x], acc.at[dsth_v.at[0, 0]], qssem[x])

        def gather_start(kc, p):
            for u in range(SUB):
                pltpu.async_copy(
                    support_hbm.at[
                        src_v.at[pl.ds(kc * CHUNK + u * SUBR, SUBR)]],
                    rowsb[p].at[pl.ds(u * SUBR, SUBR)], gsem[p * SUB + u])

        def gather_sub_wait(kc, p, u):
            pltpu.make_async_copy(
                support_hbm.at[
                    src_v.at[pl.ds(kc * CHUNK + u * SUBR, SUBR)]],
                rowsb[p].at[pl.ds(u * SUBR, SUBR)],
                gsem[p * SUB + u]).wait()

        def sup_body(sup, carry):
            @pl.when(sup > 0)
            def _():
                half_scatter_desc(0).wait()
                half_scatter_desc(1).wait()
            ebase = (cbase + sup * SUPER) * CHUNK
            h1 = pltpu.async_copy(
                src_hbm.at[pl.ds(ebase, SEDGES)], src_v, isem[0])
            h2 = pltpu.async_copy(
                dsth_hbm.at[pl.ds((cbase + sup * SUPER) * 2, SUPER * 2)],
                dsth_v, isem[1])
            h3 = pltpu.async_copy(
                vals_hbm.at[pl.ds(ebase, SEDGES)],
                vals_v.at[pl.ds(0, SEDGES)], isem[2])
            h1.wait()
            h2.wait()
            h3.wait()
            gather_start(0, 0)

            for kk in range(SUPER):
                p = kk % NBUF
                if kk + 1 < SUPER:
                    gather_start(kk + 1, (kk + 1) % NBUF)
                for h in range(2):
                    gather_sub_wait(kk, p, 2 * h)
                    gather_sub_wait(kk, p, 2 * h + 1)
                    if kk >= 1:
                        half_scatter_desc(h).wait()

                    def scale(j8, inner):
                        base = kk * CHUNK + h * HROWS + j8 * 8
                        val16 = vals_v[pl.ds(base, LANES)]
                        for e in range(8):
                            vj = lax.gather(
                                val16, jnp.full((LANES, 1), e, jnp.int32),
                                lax.GatherDimensionNumbers(
                                    offset_dims=(),
                                    collapsed_slice_dims=(0,),
                                    start_index_map=(0,)),
                                (1,),
                                mode=lax.GatherScatterMode.PROMISE_IN_BOUNDS)
                            j = h * HROWS + j8 * 8 + e
                            jh = j8 * 8 + e
                            for g in range(D // 32):
                                pi = rowsb[p][j, pl.ds(LANES * g, LANES)]
                                va = plsc.bitcast(pi << 16, jnp.float32)
                                vb = plsc.bitcast(
                                    pi & jnp.int32(-65536), jnp.float32)
                                hstag[h][jh, pl.ds(32 * g, LANES)] = (
                                    va * vj)
                                hstag[h][jh, pl.ds(32 * g + LANES,
                                                   LANES)] = vb * vj
                        return inner
                    lax.fori_loop(0, HROWS // 8, scale, 0)

                    pltpu.async_copy(
                        hstag[h], acc.at[dsth_v.at[kk * 2 + h, 0]],
                        qssem[h], add=True)
            return carry
        lax.fori_loop(0, n_super, sup_body, 0)
        half_scatter_desc(0).wait()
        half_scatter_desc(1).wait()
        plsc.subcore_barrier()

        pltpu.sync_copy(acc.at[pl.ds(row0, rows_per_tile)],
                        out_hbm.at[c, pl.ds(row0, rows_per_tile)])

    return k(support, src, dsth, vals)


def kernel(x, edge_index, edge_vals, W, b):
    N = x.shape[0]
    E = edge_vals.shape[0]
    support_bf = _matmul_bf16(x, W[:, jnp.array(_PCOL)])
    support = lax.bitcast_convert_type(
        support_bf.reshape(N, D // 2, 2), jnp.int32)

    e_per_w = ((E + NW - 1) // NW + SEDGES - 1) // SEDGES * SEDGES
    pad = e_per_w * NW - E
    src = edge_index[0]
    dst = edge_index[1]
    vals = edge_vals
    if pad:
        fill = jnp.arange(pad, dtype=jnp.int32) % N
        src = jnp.concatenate([src, fill])
        dst = jnp.concatenate([dst, fill])
        vals = jnp.concatenate([vals, jnp.zeros((pad,), vals.dtype)])
    dsth = dst.reshape(-1, 1, HROWS)

    partials = _spmm(support, src, dsth, vals, e_per_w)
    return _finalize(partials, b, N)


# speedup vs baseline: 1.6466x; 1.6466x over previous
import functools

import jax
import jax.numpy as jnp
from jax import lax
from jax.experimental import pallas as pl
from jax.experimental.pallas import tpu as pltpu
from jax.experimental.pallas import tpu_sc as plsc

D = 128
NC = 2
NS = 16
NW = NC * NS
CHUNK = 128
LANES = 16


def _mm_body(x_ref, w_ref, o_ref):
    o_ref[...] = jnp.dot(x_ref[...], w_ref[...],
                         preferred_element_type=jnp.float32)


def _matmul(x, W):
    M = x.shape[0]
    BM = 1000
    return pl.pallas_call(
        _mm_body,
        grid=(M // BM,),
        in_specs=[pl.BlockSpec((BM, D), lambda i: (i, 0)),
                  pl.BlockSpec((D, D), lambda i: (0, 0))],
        out_specs=pl.BlockSpec((BM, D), lambda i: (i, 0)),
        out_shape=jax.ShapeDtypeStruct((M, D), jnp.float32),
    )(x, W)


def _fin_body(p_ref, b_ref, o_ref):
    o_ref[...] = jnp.maximum(p_ref[0] + p_ref[1] + b_ref[...], 0.0)


def _finalize(partials, b, N):
    BM = 1000
    return pl.pallas_call(
        _fin_body,
        grid=(N // BM,),
        in_specs=[pl.BlockSpec((2, BM, D), lambda i: (0, i, 0)),
                  pl.BlockSpec((1, D), lambda i: (0, 0))],
        out_specs=pl.BlockSpec((BM, D), lambda i: (i, 0)),
        out_shape=jax.ShapeDtypeStruct((N, D), jnp.float32),
    )(partials, b.reshape(1, D))


NBUF = 2
SUB = 4
SUBR = CHUNK // SUB
SUPER = 8
SEDGES = SUPER * CHUNK


def _spmm(support, src, dst3, vals, e_per_w):
    N = support.shape[0]
    n_chunks = e_per_w // CHUNK
    n_super = n_chunks // SUPER
    rows_per_tile = ((N + NS - 1) // NS + 7) // 8 * 8
    NP = rows_per_tile * NS

    mesh = plsc.VectorSubcoreMesh(core_axis_name="c", subcore_axis_name="s")

    @functools.partial(
        pl.kernel,
        mesh=mesh,
        out_type=jax.ShapeDtypeStruct((NC, NP, D), jnp.float32),
        scratch_types=[
            pltpu.VMEM_SHARED((NP, D), jnp.float32),
            pltpu.VMEM((SEDGES,), jnp.int32),
            pltpu.VMEM((SUPER, 1, CHUNK), jnp.int32),
            pltpu.VMEM((SEDGES,), jnp.float32),
        ] + [pltpu.VMEM((CHUNK, D), jnp.float32) for _ in range(NBUF)]
          + [pltpu.SemaphoreType.DMA
             for _ in range(NBUF * SUB + NBUF + 3)],
    )
    def k(support_hbm, src_hbm, dst3_hbm, vals_hbm, out_hbm,
          acc, src_v, dst_v, vals_v, *bufs_sems):
        rows = bufs_sems[:NBUF]
        gsem = bufs_sems[NBUF:NBUF + NBUF * SUB]
        ssem = bufs_sems[NBUF + NBUF * SUB:NBUF + NBUF * SUB + NBUF]
        isem = bufs_sems[NBUF + NBUF * SUB + NBUF:]
        c = lax.axis_index("c")
        s = lax.axis_index("s")
        wid = c * NS + s

        def zrow(r, carry):
            for g in range(D // LANES):
                rows[0][r, pl.ds(g * LANES, LANES)] = jnp.zeros(
                    (LANES,), jnp.float32)
            return carry
        lax.fori_loop(0, CHUNK, zrow, 0)
        row0 = s * rows_per_tile
        nfull = rows_per_tile // CHUNK
        rem = rows_per_tile % CHUNK
        for i in range(nfull):
            pltpu.sync_copy(rows[0], acc.at[pl.ds(row0 + i * CHUNK, CHUNK)])
        if rem:
            pltpu.sync_copy(rows[0].at[pl.ds(0, rem)],
                            acc.at[pl.ds(row0 + nfull * CHUNK, rem)])
        plsc.subcore_barrier()

        cbase = wid * n_chunks

        def scatter_desc(p, ksel):
            return pltpu.make_async_copy(
                rows[p], acc.at[dst_v.at[ksel, 0]], ssem[p])

        def gather_start(kc, p):
            for u in range(SUB):
                pltpu.async_copy(
                    support_hbm.at[
                        src_v.at[pl.ds(kc * CHUNK + u * SUBR, SUBR)]],
                    rows[p].at[pl.ds(u * SUBR, SUBR)], gsem[p * SUB + u])

        def gather_wait(kc, p):
            for u in range(SUB):
                pltpu.make_async_copy(
                    support_hbm.at[
                        src_v.at[pl.ds(kc * CHUNK + u * SUBR, SUBR)]],
                    rows[p].at[pl.ds(u * SUBR, SUBR)],
                    gsem[p * SUB + u]).wait()

        def sup_body(sup, carry):
            @pl.when(sup > 0)
            def _():
                for p in range(NBUF):
                    scatter_desc(p, 0).wait()
            ebase = (cbase + sup * SUPER) * CHUNK
            h1 = pltpu.async_copy(
                src_hbm.at[pl.ds(ebase, SEDGES)], src_v, isem[0])
            h2 = pltpu.async_copy(
                dst3_hbm.at[pl.ds(cbase + sup * SUPER, SUPER)], dst_v,
                isem[1])
            h3 = pltpu.async_copy(
                vals_hbm.at[pl.ds(ebase, SEDGES)], vals_v, isem[2])
            h1.wait()
            h2.wait()
            h3.wait()
            for t in range(min(NBUF - 1, SUPER)):
                gather_start(t, t)

            for kk in range(SUPER):
                p = kk % NBUF
                q = (kk + NBUF - 1) % NBUF
                gather_wait(kk, p)

                def scale(j16, inner):
                    val16 = vals_v[pl.ds(kk * CHUNK + j16 * LANES, LANES)]
                    for l in range(LANES):
                        vj = lax.gather(
                            val16, jnp.full((LANES, 1), l, jnp.int32),
                            lax.GatherDimensionNumbers(
                                offset_dims=(), collapsed_slice_dims=(0,),
                                start_index_map=(0,)),
                            (1,),
                            mode=lax.GatherScatterMode.PROMISE_IN_BOUNDS)
                        j = j16 * LANES + l
                        for g in range(D // LANES):
                            rv = rows[p][j, pl.ds(g * LANES, LANES)]
                            rows[p][j, pl.ds(g * LANES, LANES)] = rv * vj
                    return inner
                lax.fori_loop(0, CHUNK // LANES, scale, 0)

                pltpu.async_copy(rows[p], acc.at[dst_v.at[kk, 0]],
                                 ssem[p], add=True)
                if kk + NBUF - 1 < SUPER:
                    if kk >= 1:
                        scatter_desc(q, 0).wait()
                    gather_start(kk + NBUF - 1, q)
            return carry
        lax.fori_loop(0, n_super, sup_body, 0)
        for p in range(NBUF):
            scatter_desc(p, 0).wait()
        plsc.subcore_barrier()

        pltpu.sync_copy(acc.at[pl.ds(row0, rows_per_tile)],
                        out_hbm.at[c, pl.ds(row0, rows_per_tile)])

    return k(support, src, dst3, vals)


def kernel(x, edge_index, edge_vals, W, b):
    N = x.shape[0]
    E = edge_vals.shape[0]
    support = _matmul(x, W)

    e_per_w = ((E + NW - 1) // NW + SEDGES - 1) // SEDGES * SEDGES
    pad = e_per_w * NW - E
    src = edge_index[0]
    dst = edge_index[1]
    vals = edge_vals
    if pad:
        fill = jnp.arange(pad, dtype=jnp.int32) % N
        src = jnp.concatenate([src, fill])
        dst = jnp.concatenate([dst, fill])
        vals = jnp.concatenate([vals, jnp.zeros((pad,), vals.dtype)])
    dst3 = dst.reshape(-1, 1, CHUNK)

    partials = _spmm(support, src, dst3, vals, e_per_w)
    return _finalize(partials, b, N)


# speedup vs baseline: 1.6553x; 1.0053x over previous
import functools

import jax
import jax.numpy as jnp
from jax import lax
from jax.experimental import pallas as pl
from jax.experimental.pallas import tpu as pltpu
from jax.experimental.pallas import tpu_sc as plsc

D = 128
NC = 2
NS = 16
NW = NC * NS
CHUNK = 128
LANES = 16


def _mm_body(x_ref, w_ref, o_ref):
    o_ref[...] = jnp.dot(x_ref[...], w_ref[...],
                         preferred_element_type=jnp.float32)


def _matmul(x, W):
    M = x.shape[0]
    BM = 1000
    return pl.pallas_call(
        _mm_body,
        grid=(M // BM,),
        in_specs=[pl.BlockSpec((BM, D), lambda i: (i, 0)),
                  pl.BlockSpec((D, D), lambda i: (0, 0))],
        out_specs=pl.BlockSpec((BM, D), lambda i: (i, 0)),
        out_shape=jax.ShapeDtypeStruct((M, D), jnp.float32),
    )(x, W)


def _fin_body(p_ref, b_ref, o_ref):
    o_ref[...] = jnp.maximum(p_ref[0] + p_ref[1] + b_ref[...], 0.0)


def _finalize(partials, b, N):
    BM = 1000
    return pl.pallas_call(
        _fin_body,
        grid=(N // BM,),
        in_specs=[pl.BlockSpec((2, BM, D), lambda i: (0, i, 0)),
                  pl.BlockSpec((1, D), lambda i: (0, 0))],
        out_specs=pl.BlockSpec((BM, D), lambda i: (i, 0)),
        out_shape=jax.ShapeDtypeStruct((N, D), jnp.float32),
    )(partials, b.reshape(1, D))


NBUF = 2
SUB = 4
SUBR = CHUNK // SUB
SUPER = 8
SEDGES = SUPER * CHUNK


def _spmm(support, src, dst3, vals, e_per_w):
    N = support.shape[0]
    n_chunks = e_per_w // CHUNK
    n_super = n_chunks // SUPER
    rows_per_tile = ((N + NS - 1) // NS + 7) // 8 * 8
    NP = rows_per_tile * NS

    mesh = plsc.VectorSubcoreMesh(core_axis_name="c", subcore_axis_name="s")

    @functools.partial(
        pl.kernel,
        mesh=mesh,
        out_type=jax.ShapeDtypeStruct((NC, NP, D), jnp.float32),
        scratch_types=[
            pltpu.VMEM_SHARED((NP, D), jnp.float32),
            pltpu.VMEM((SEDGES,), jnp.int32),
            pltpu.VMEM((SUPER, 1, CHUNK), jnp.int32),
            pltpu.VMEM((SEDGES,), jnp.float32),
        ] + [pltpu.VMEM((CHUNK, D), jnp.float32) for _ in range(NBUF)]
          + [pltpu.SemaphoreType.DMA
             for _ in range(NBUF * SUB + NBUF + 3)],
    )
    def k(support_hbm, src_hbm, dst3_hbm, vals_hbm, out_hbm,
          acc, src_v, dst_v, vals_v, *bufs_sems):
        rows = bufs_sems[:NBUF]
        gsem = bufs_sems[NBUF:NBUF + NBUF * SUB]
        ssem = bufs_sems[NBUF + NBUF * SUB:NBUF + NBUF * SUB + NBUF]
        isem = bufs_sems[NBUF + NBUF * SUB + NBUF:]
        c = lax.axis_index("c")
        s = lax.axis_index("s")
        wid = c * NS + s

        def zrow(r, carry):
            for g in range(D // LANES):
                rows[0][r, pl.ds(g * LANES, LANES)] = jnp.zeros(
                    (LANES,), jnp.float32)
            return carry
        lax.fori_loop(0, CHUNK, zrow, 0)
        row0 = s * rows_per_tile
        nfull = rows_per_tile // CHUNK
        rem = rows_per_tile % CHUNK
        zh = []
        for i in range(nfull):
            zh.append(pltpu.async_copy(
                rows[0], acc.at[pl.ds(row0 + i * CHUNK, CHUNK)], gsem[i]))
        if rem:
            zh.append(pltpu.async_copy(
                rows[0].at[pl.ds(0, rem)],
                acc.at[pl.ds(row0 + nfull * CHUNK, rem)], gsem[nfull]))
        for hcopy in zh:
            hcopy.wait()
        plsc.subcore_barrier()

        cbase = wid * n_chunks

        def scatter_desc(p, ksel):
            return pltpu.make_async_copy(
                rows[p], acc.at[dst_v.at[ksel, 0]], ssem[p])

        def gather_start(kc, p):
            for u in range(SUB):
                pltpu.async_copy(
                    support_hbm.at[
                        src_v.at[pl.ds(kc * CHUNK + u * SUBR, SUBR)]],
                    rows[p].at[pl.ds(u * SUBR, SUBR)], gsem[p * SUB + u])

        def gather_wait(kc, p):
            for u in range(SUB):
                pltpu.make_async_copy(
                    support_hbm.at[
                        src_v.at[pl.ds(kc * CHUNK + u * SUBR, SUBR)]],
                    rows[p].at[pl.ds(u * SUBR, SUBR)],
                    gsem[p * SUB + u]).wait()

        def sup_body(sup, carry):
            ebase = (cbase + sup * SUPER) * CHUNK
            h1 = pltpu.async_copy(
                src_hbm.at[pl.ds(ebase, SEDGES)], src_v, isem[0])
            h3 = pltpu.async_copy(
                vals_hbm.at[pl.ds(ebase, SEDGES)], vals_v, isem[2])

            @pl.when(sup > 0)
            def _():
                for p in range(NBUF):
                    scatter_desc(p, 0).wait()
            h2 = pltpu.async_copy(
                dst3_hbm.at[pl.ds(cbase + sup * SUPER, SUPER)], dst_v,
                isem[1])
            h1.wait()
            h2.wait()
            h3.wait()
            for t in range(min(NBUF - 1, SUPER)):
                gather_start(t, t)

            for kk in range(SUPER):
                p = kk % NBUF
                q = (kk + NBUF - 1) % NBUF
                gather_wait(kk, p)

                def scale(j16, inner):
                    val16 = vals_v[pl.ds(kk * CHUNK + j16 * LANES, LANES)]
                    for l in range(LANES):
                        vj = lax.gather(
                            val16, jnp.full((LANES, 1), l, jnp.int32),
                            lax.GatherDimensionNumbers(
                                offset_dims=(), collapsed_slice_dims=(0,),
                                start_index_map=(0,)),
                            (1,),
                            mode=lax.GatherScatterMode.PROMISE_IN_BOUNDS)
                        j = j16 * LANES + l
                        for g in range(D // LANES):
                            rv = rows[p][j, pl.ds(g * LANES, LANES)]
                            rows[p][j, pl.ds(g * LANES, LANES)] = rv * vj
                    return inner
                lax.fori_loop(0, CHUNK // LANES, scale, 0)

                pltpu.async_copy(rows[p], acc.at[dst_v.at[kk, 0]],
                                 ssem[p], add=True)
                if kk + NBUF - 1 < SUPER:
                    if kk >= 1:
                        scatter_desc(q, 0).wait()
                    gather_start(kk + NBUF - 1, q)
            return carry
        lax.fori_loop(0, n_super, sup_body, 0)
        for p in range(NBUF):
            scatter_desc(p, 0).wait()
        plsc.subcore_barrier()

        pltpu.sync_copy(acc.at[pl.ds(row0, rows_per_tile)],
                        out_hbm.at[c, pl.ds(row0, rows_per_tile)])

    return k(support, src, dst3, vals)


def kernel(x, edge_index, edge_vals, W, b):
    N = x.shape[0]
    E = edge_vals.shape[0]
    support = _matmul(x, W)

    e_per_w = ((E + NW - 1) // NW + SEDGES - 1) // SEDGES * SEDGES
    pad = e_per_w * NW - E
    src = edge_index[0]
    dst = edge_index[1]
    vals = edge_vals
    if pad:
        fill = jnp.arange(pad, dtype=jnp.int32) % N
        src = jnp.concatenate([src, fill])
        dst = jnp.concatenate([dst, fill])
        vals = jnp.concatenate([vals, jnp.zeros((pad,), vals.dtype)])
    dst3 = dst.reshape(-1, 1, CHUNK)

    partials = _spmm(support, src, dst3, vals, e_per_w)
    return _finalize(partials, b, N)
